# baseline (speedup 1.0000x reference)
import functools

import jax
import jax.numpy as jnp
import numpy as np
from jax import lax
from jax.experimental import pallas as pl
from jax.experimental.pallas import tpu as pltpu
from jax.experimental.pallas import tpu_sc as plsc

V = 10000
E = 320000
DN = 128
DE = 16
H = 256
HH = H // 2

N_TILES = 16
EDGES_PER_TILE = E // N_TILES
CHUNK = 80
N_CHUNKS = EDGES_PER_TILE // CHUNK
OWN_ROWS = 640
OWN_CHUNKS_FULL = OWN_ROWS // CHUNK
OWN_CHUNKS_LAST = (V - 15 * OWN_ROWS) // CHUNK

_f32 = jnp.float32
_bf16 = jnp.bfloat16

def _pack_cols(half, hi):
    return np.concatenate(
        [128 * half + 32 * g + 16 * hi + np.arange(16) for g in range(4)])

LO0, HI0 = _pack_cols(0, 0), _pack_cols(0, 1)
LO1, HI1 = _pack_cols(1, 0), _pack_cols(1, 1)


def _silu(x):
    return x * (1.0 / (1.0 + jnp.exp(-x)))


def _embed_body(x_ref, wn_ref, bn_ref, wm_ref, bm_ref, h_ref, hm_ref):
    h = _silu(jnp.dot(x_ref[...], wn_ref[...],
                      preferred_element_type=_f32) + bn_ref[...])
    h_ref[...] = h
    hm = jnp.dot(h, wm_ref[...], preferred_element_type=_f32) + bm_ref[...]
    hm_ref[0] = hm[:, :HH]
    hm_ref[1] = hm[:, HH:]


def _node_embed(x, Wn, bn, Wm, bm):
    VB = 2000
    grid = (V // VB,)
    return pl.pallas_call(
        _embed_body,
        grid=grid,
        in_specs=[
            pl.BlockSpec((VB, DN), lambda i: (i, 0)),
            pl.BlockSpec((DN, H), lambda i: (0, 0)),
            pl.BlockSpec((1, H), lambda i: (0, 0)),
            pl.BlockSpec((H, H), lambda i: (0, 0)),
            pl.BlockSpec((1, H), lambda i: (0, 0)),
        ],
        out_specs=[
            pl.BlockSpec((VB, H), lambda i: (i, 0)),
            pl.BlockSpec((2, VB, HH), lambda i: (0, i, 0)),
        ],
        out_shape=[
            jax.ShapeDtypeStruct((V, H), _f32),
            jax.ShapeDtypeStruct((2, V, HH), _f32),
        ],
    )(x, Wn, bn.reshape(1, H), Wm, bm.reshape(1, H))


def _bf16_bits(x):
    return jax.lax.bitcast_convert_type(
        x.astype(_bf16).astype(_f32), jnp.uint32)


def _edge_body(ea_ref, we_ref, be_ref, wlo0_ref, whi0_ref, wlo1_ref,
               whi1_ref, em_ref):
    e = _silu(jnp.dot(ea_ref[...], we_ref[...],
                      preferred_element_type=_f32) + be_ref[...])
    eb = e.astype(_bf16)
    for h, (wlo, whi) in enumerate(((wlo0_ref, whi0_ref),
                                    (wlo1_ref, whi1_ref))):
        lo = jnp.dot(eb, wlo[...], preferred_element_type=_f32)
        hi = jnp.dot(eb, whi[...], preferred_element_type=_f32)
        em_ref[h] = ((_bf16_bits(hi) & jnp.uint32(0xFFFF0000)) | (
            _bf16_bits(lo) >> 16)).astype(jnp.int32)


def _edge_messages(ea, We, be, Wlo0, Whi0, Wlo1, Whi1):
    CE = 2000
    grid = (E // CE,)
    return pl.pallas_call(
        _edge_body,
        grid=grid,
        in_specs=[
            pl.BlockSpec((CE, DE), lambda i: (i, 0)),
            pl.BlockSpec((DE, H), lambda i: (0, 0)),
            pl.BlockSpec((1, H), lambda i: (0, 0)),
            pl.BlockSpec((H, HH // 2), lambda i: (0, 0)),
            pl.BlockSpec((H, HH // 2), lambda i: (0, 0)),
            pl.BlockSpec((H, HH // 2), lambda i: (0, 0)),
            pl.BlockSpec((H, HH // 2), lambda i: (0, 0)),
        ],
        out_specs=pl.BlockSpec((2, CE, HH // 2), lambda i: (0, i, 0)),
        out_shape=jax.ShapeDtypeStruct((2, E, HH // 2), jnp.int32),
    )(ea, We, be.reshape(1, H), Wlo0, Whi0, Wlo1, Whi1)


def _sc_body(hm_hbm, em_hbm, src_hbm, dst_hbm, out_hbm,
             rows0, rows1, em0, em1,
             isrc0, isrc1, isrc2, isrc3, idst0, idst1, idst2, idst3,
             igat0, igat1,
             gsem0, gsem1, esem0, esem1, ssem0, ssem1,
             isem0, isem1, isem2, isem3, agg_sh):
    c = lax.axis_index("c")
    s = lax.axis_index("s")
    n_own = jnp.where(s == N_TILES - 1, OWN_CHUNKS_LAST, OWN_CHUNKS_FULL)
    cV = c * V
    cE = c * E
    base_t = s * EDGES_PER_TILE
    rows = (rows0, rows1)
    ems = (em0, em1)
    igat = (igat0, igat1)
    isrc = (isrc0, isrc1, isrc2, isrc3)
    idst = (idst0, idst1, idst2, idst3)
    gsem = (gsem0, gsem1)
    esem = (esem0, esem1)
    ssem = (ssem0, ssem1)
    isem = (isem0, isem1, isem2, isem3)

    def _zrow(r, carry):
        for k in range(HH // 16):
            rows0[r, pl.ds(k * 16, 16)] = jnp.zeros((16,), _f32)
        return carry
    lax.fori_loop(0, CHUNK, _zrow, 0)

    def _zcopy(j, carry):
        pltpu.sync_copy(rows0,
                        agg_sh.at[pl.ds(s * OWN_ROWS + j * CHUNK, CHUNK)])
        return carry
    lax.fori_loop(0, n_own, _zcopy, 0)

    def _start_idx(ci, sl4):
        eb = pl.ds(base_t + ci * CHUNK, CHUNK)
        pltpu.async_copy(src_hbm.at[eb], isrc[sl4], isem[sl4])
        pltpu.async_copy(dst_hbm.at[eb], idst[sl4], isem[sl4])

    def _wait_idx(ci, sl4):
        eb = pl.ds(base_t + ci * CHUNK, CHUNK)
        pltpu.make_async_copy(src_hbm.at[eb], isrc[sl4], isem[sl4]).wait()
        pltpu.make_async_copy(dst_hbm.at[eb], idst[sl4], isem[sl4]).wait()

    def _build_igat(sl4, b2):
        for k in range(CHUNK // 16):
            sl = pl.ds(k * 16, 16)
            igat[b2][sl] = isrc[sl4][sl] + cV

    def _start_in(ci, b2):
        pltpu.async_copy(hm_hbm.at[igat[b2]], rows[b2], gsem[b2])
        pltpu.async_copy(em_hbm.at[pl.ds(cE + base_t + ci * CHUNK, CHUNK)],
                         ems[b2], esem[b2])

    def _wait_in(ci, b2):
        pltpu.make_async_copy(hm_hbm.at[igat[b2]], rows[b2],
                              gsem[b2]).wait()
        pltpu.make_async_copy(em_hbm.at[pl.ds(cE + base_t + ci * CHUNK,
                                              CHUNK)],
                              ems[b2], esem[b2]).wait()

    def _wait_scat(b2, sl4):
        pltpu.make_async_copy(rows[b2], agg_sh.at[idst[sl4]],
                              ssem[b2]).wait()

    _start_idx(0, 0)
    _start_idx(1, 1)
    _start_idx(2, 2)
    _wait_idx(0, 0)
    _build_igat(0, 0)
    _start_in(0, 0)
    plsc.subcore_barrier()

    def _step(ci, b2, sl4):
        nsl = (sl4 + 1) % 4
        @pl.when(ci + 1 < N_CHUNKS)
        def _():
            _wait_idx(ci + 1, nsl)
            _build_igat(nsl, 1 - b2)
            @pl.when(ci >= 1)
            def _():
                _wait_scat(1 - b2, (sl4 + 3) % 4)
            _start_in(ci + 1, 1 - b2)
        @pl.when(ci + 3 < N_CHUNKS)
        def _():
            _start_idx(ci + 3, (sl4 + 3) % 4)
        _wait_in(ci, b2)

        @plsc.parallel_loop(0, CHUNK, unroll=4)
        def _row(r):
            for g in range(HH // 32):
                w = ems[b2][r, pl.ds(g * 16, 16)]
                flo = jax.lax.bitcast_convert_type(w << 16, _f32)
                fhi = jax.lax.bitcast_convert_type(w & jnp.int32(-65536),
                                                   _f32)
                for t, fv in ((0, flo), (1, fhi)):
                    sl = pl.ds(g * 32 + t * 16, 16)
                    u = rows[b2][r, sl] + fv
                    rows[b2][r, sl] = u * (1.0 / (1.0 + jnp.exp(-u)))
        pltpu.async_copy(rows[b2], agg_sh.at[idst[sl4]], ssem[b2],
                         add=True)

    def _quad(i, carry):
        ci = 4 * i
        _step(ci, 0, 0)
        _step(ci + 1, 1, 1)
        _step(ci + 2, 0, 2)
        _step(ci + 3, 1, 3)
        return carry
    lax.fori_loop(0, N_CHUNKS // 4, _quad, 0)
    _step(N_CHUNKS - 2, 0, 0)
    _step(N_CHUNKS - 1, 1, 1)

    _wait_scat(0, 0)
    _wait_scat(1, 1)
    plsc.subcore_barrier()

    def _wcopy(j, carry):
        off = s * OWN_ROWS + j * CHUNK
        pltpu.sync_copy(agg_sh.at[pl.ds(off, CHUNK)],
                        out_hbm.at[pl.ds(cV + off, CHUNK)])
        return carry
    lax.fori_loop(0, n_own, _wcopy, 0)


def _sc_aggregate(hm2, em2, src3, dst3):
    mesh = plsc.VectorSubcoreMesh(core_axis_name="c", subcore_axis_name="s")
    f = functools.partial(
        pl.kernel,
        mesh=mesh,
        out_type=jax.ShapeDtypeStruct((2 * V, HH), _f32),
        scratch_types=(
            [pltpu.VMEM((CHUNK, HH), _f32)] * 2
            + [pltpu.VMEM((CHUNK, HH // 2), jnp.int32)] * 2
            + [pltpu.VMEM((CHUNK,), jnp.int32)] * 10
            + [pltpu.SemaphoreType.DMA] * 10
            + [pltpu.VMEM_SHARED((V, HH), _f32)]
        ),
    )(_sc_body)
    return f(hm2, em2, src3, dst3)


def _update_body(h_ref, alo_ref, ahi_ref, wu_ref, bu_ref, g_ref, bt_ref,
                 wm_ref, bm_ref, h1_ref, hm_ref):
    upd = (jnp.dot(alo_ref[0], wu_ref[...][:HH, :],
                   preferred_element_type=_f32)
           + jnp.dot(ahi_ref[0], wu_ref[...][HH:, :],
                     preferred_element_type=_f32)
           + bu_ref[...])
    y = h_ref[...] + _silu(upd)
    mu = jnp.mean(y, axis=-1, keepdims=True)
    yc = y - mu
    var = jnp.mean(yc * yc, axis=-1, keepdims=True)
    h1 = yc * jax.lax.rsqrt(var + 1e-5) * g_ref[...] + bt_ref[...]
    h1_ref[...] = h1
    if hm_ref is not None:
        hm = jnp.dot(h1, wm_ref[...], preferred_element_type=_f32) + bm_ref[...]
        hm_ref[0] = hm[:, :HH]
        hm_ref[1] = hm[:, HH:]


def _update(h, agg2, Wu, bu, g, bt, Wm=None, bm=None):
    VB = 2000
    grid = (V // VB,)
    with_next = Wm is not None
    if not with_next:
        Wm = jnp.zeros((H, H), _f32)
        bm = jnp.zeros((H,), _f32)
    agg3 = agg2.reshape(2, V, HH)

    def body(h_ref, alo_ref, ahi_ref, wu_ref, bu_ref, g_ref, bt_ref,
             wm_ref, bm_ref, h1_ref, *maybe_hm):
        _update_body(h_ref, alo_ref, ahi_ref, wu_ref, bu_ref, g_ref, bt_ref,
                     wm_ref, bm_ref, h1_ref,
                     maybe_hm[0] if maybe_hm else None)

    out_specs = [pl.BlockSpec((VB, H), lambda i: (i, 0))]
    out_shape = [jax.ShapeDtypeStruct((V, H), _f32)]
    if with_next:
        out_specs.append(pl.BlockSpec((2, VB, HH), lambda i: (0, i, 0)))
        out_shape.append(jax.ShapeDtypeStruct((2, V, HH), _f32))

    res = pl.pallas_call(
        body,
        grid=grid,
        in_specs=[
            pl.BlockSpec((VB, H), lambda i: (i, 0)),
            pl.BlockSpec((1, VB, HH), lambda i: (0, i, 0)),
            pl.BlockSpec((1, VB, HH), lambda i: (1, i, 0)),
            pl.BlockSpec((H, H), lambda i: (0, 0)),
            pl.BlockSpec((1, H), lambda i: (0, 0)),
            pl.BlockSpec((1, H), lambda i: (0, 0)),
            pl.BlockSpec((1, H), lambda i: (0, 0)),
            pl.BlockSpec((H, H), lambda i: (0, 0)),
            pl.BlockSpec((1, H), lambda i: (0, 0)),
        ],
        out_specs=out_specs,
        out_shape=out_shape,
    )(h, agg3, agg3, Wu, bu.reshape(1, H), g.reshape(1, H), bt.reshape(1, H),
      Wm, bm.reshape(1, H))
    return res if with_next else (res[0], None)


def kernel(x, edge_index, edge_attr, W_node, b_node, W_edge, b_edge,
           W_m0, b_m0, W_u0, b_u0, ln_g0, ln_b0,
           W_m1, b_m1, W_u1, b_u1, ln_g1, ln_b1):
    src3 = edge_index[0]
    dst3 = edge_index[1]

    def _wm_sel(Wm):
        return tuple(Wm[:, sel].astype(_bf16) for sel in (LO0, HI0,
                                                          LO1, HI1))
    wm0s = _wm_sel(W_m0)
    wm1s = _wm_sel(W_m1)

    h, hm0 = _node_embed(x, W_node, b_node, W_m0, b_m0)
    em0 = _edge_messages(edge_attr, W_edge, b_edge, *wm0s)

    agg0 = _sc_aggregate(hm0.reshape(2 * V, HH),
                         em0.reshape(2 * E, HH // 2), src3, dst3)
    em1 = _edge_messages(edge_attr, W_edge, b_edge, *wm1s)
    h1, hm1 = _update(h, agg0, W_u0, b_u0, ln_g0, ln_b0, W_m1, b_m1)

    agg1 = _sc_aggregate(hm1.reshape(2 * V, HH),
                         em1.reshape(2 * E, HH // 2), src3, dst3)
    h2, _ = _update(h1, agg1, W_u1, b_u1, ln_g1, ln_b1)
    return h2


# speedup vs baseline: 1.0385x; 1.0385x over previous
import functools

import jax
import jax.numpy as jnp
import numpy as np
from jax import lax
from jax.experimental import pallas as pl
from jax.experimental.pallas import tpu as pltpu
from jax.experimental.pallas import tpu_sc as plsc

V = 10000
E = 320000
DN = 128
DE = 16
H = 256
HH = H // 2

N_TILES = 16
EDGES_PER_TILE = E // N_TILES
CHUNK = 80
N_CHUNKS = EDGES_PER_TILE // CHUNK
OWN_ROWS = 640
OWN_CHUNKS_FULL = OWN_ROWS // CHUNK
OWN_CHUNKS_LAST = (V - 15 * OWN_ROWS) // CHUNK

_f32 = jnp.float32
_bf16 = jnp.bfloat16

def _pack_cols(half, hi):
    return np.concatenate(
        [128 * half + 32 * g + 16 * hi + np.arange(16) for g in range(4)])

LO0, HI0 = _pack_cols(0, 0), _pack_cols(0, 1)
LO1, HI1 = _pack_cols(1, 0), _pack_cols(1, 1)


def _silu(x):
    return x * (1.0 / (1.0 + jnp.exp(-x)))


def _embed_body(x_ref, wn_ref, bn_ref, wm_ref, bm_ref, h_ref, hm_ref):
    h = _silu(jnp.dot(x_ref[...], wn_ref[...],
                      preferred_element_type=_f32) + bn_ref[...])
    h_ref[...] = h
    hm = jnp.dot(h, wm_ref[...], preferred_element_type=_f32) + bm_ref[...]
    hm_ref[0] = hm[:, :HH]
    hm_ref[1] = hm[:, HH:]


def _node_embed(x, Wn, bn, Wm, bm):
    VB = 2000
    grid = (V // VB,)
    return pl.pallas_call(
        _embed_body,
        grid=grid,
        in_specs=[
            pl.BlockSpec((VB, DN), lambda i: (i, 0)),
            pl.BlockSpec((DN, H), lambda i: (0, 0)),
            pl.BlockSpec((1, H), lambda i: (0, 0)),
            pl.BlockSpec((H, H), lambda i: (0, 0)),
            pl.BlockSpec((1, H), lambda i: (0, 0)),
        ],
        out_specs=[
            pl.BlockSpec((VB, H), lambda i: (i, 0)),
            pl.BlockSpec((2, VB, HH), lambda i: (0, i, 0)),
        ],
        out_shape=[
            jax.ShapeDtypeStruct((V, H), _f32),
            jax.ShapeDtypeStruct((2, V, HH), _f32),
        ],
    )(x, Wn, bn.reshape(1, H), Wm, bm.reshape(1, H))


def _bf16_bits(x):
    return jax.lax.bitcast_convert_type(
        x.astype(_bf16).astype(_f32), jnp.uint32)


def _edge_body(ea_ref, we_ref, be_ref, wlo0_ref, whi0_ref, wlo1_ref,
               whi1_ref, em_ref):
    e = _silu(jnp.dot(ea_ref[...], we_ref[...],
                      preferred_element_type=_f32) + be_ref[...])
    eb = e.astype(_bf16)
    for h, (wlo, whi) in enumerate(((wlo0_ref, whi0_ref),
                                    (wlo1_ref, whi1_ref))):
        lo = jnp.dot(eb, wlo[...], preferred_element_type=_f32)
        hi = jnp.dot(eb, whi[...], preferred_element_type=_f32)
        em_ref[h] = ((_bf16_bits(hi) & jnp.uint32(0xFFFF0000)) | (
            _bf16_bits(lo) >> 16)).astype(jnp.int32)


def _edge_messages(ea, We, be, Wlo0, Whi0, Wlo1, Whi1):
    CE = 2000
    grid = (E // CE,)
    return pl.pallas_call(
        _edge_body,
        grid=grid,
        in_specs=[
            pl.BlockSpec((CE, DE), lambda i: (i, 0)),
            pl.BlockSpec((DE, H), lambda i: (0, 0)),
            pl.BlockSpec((1, H), lambda i: (0, 0)),
            pl.BlockSpec((H, HH // 2), lambda i: (0, 0)),
            pl.BlockSpec((H, HH // 2), lambda i: (0, 0)),
            pl.BlockSpec((H, HH // 2), lambda i: (0, 0)),
            pl.BlockSpec((H, HH // 2), lambda i: (0, 0)),
        ],
        out_specs=pl.BlockSpec((2, CE, HH // 2), lambda i: (0, i, 0)),
        out_shape=jax.ShapeDtypeStruct((2, E, HH // 2), jnp.int32),
    )(ea, We, be.reshape(1, H), Wlo0, Whi0, Wlo1, Whi1)


def _sc_body(hm_hbm, em_hbm, src_hbm, dst_hbm, out_hbm,
             rows0, rows1, em0, em1,
             isrc0, isrc1, isrc2, isrc3, idst0, idst1, idst2, idst3,
             igat0, igat1,
             gsem0, gsem1, esem0, esem1, ssem0, ssem1,
             isem0, isem1, isem2, isem3, agg_sh):
    c = lax.axis_index("c")
    s = lax.axis_index("s")
    n_own = jnp.where(s == N_TILES - 1, OWN_CHUNKS_LAST, OWN_CHUNKS_FULL)
    cV = c * V
    cE = c * E
    base_t = s * EDGES_PER_TILE
    rows = (rows0, rows1)
    ems = (em0, em1)
    igat = (igat0, igat1)
    isrc = (isrc0, isrc1, isrc2, isrc3)
    idst = (idst0, idst1, idst2, idst3)
    gsem = (gsem0, gsem1)
    esem = (esem0, esem1)
    ssem = (ssem0, ssem1)
    isem = (isem0, isem1, isem2, isem3)

    def _zrow(r, carry):
        for k in range(HH // 16):
            rows0[r, pl.ds(k * 16, 16)] = jnp.zeros((16,), _f32)
        return carry
    lax.fori_loop(0, CHUNK, _zrow, 0)

    def _zcopy(j, carry):
        pltpu.sync_copy(rows0,
                        agg_sh.at[pl.ds(s * OWN_ROWS + j * CHUNK, CHUNK)])
        return carry
    lax.fori_loop(0, n_own, _zcopy, 0)

    def _start_idx(ci, sl4):
        eb = pl.ds(base_t + ci * CHUNK, CHUNK)
        pltpu.async_copy(src_hbm.at[eb], isrc[sl4], isem[sl4])
        pltpu.async_copy(dst_hbm.at[eb], idst[sl4], isem[sl4])

    def _wait_idx(ci, sl4):
        eb = pl.ds(base_t + ci * CHUNK, CHUNK)
        pltpu.make_async_copy(src_hbm.at[eb], isrc[sl4], isem[sl4]).wait()
        pltpu.make_async_copy(dst_hbm.at[eb], idst[sl4], isem[sl4]).wait()

    def _build_igat(sl4, b2):
        for k in range(CHUNK // 16):
            sl = pl.ds(k * 16, 16)
            igat[b2][sl] = isrc[sl4][sl] + cV

    def _start_in(ci, b2):
        pltpu.async_copy(hm_hbm.at[igat[b2]], rows[b2], gsem[b2])
        pltpu.async_copy(em_hbm.at[pl.ds(cE + base_t + ci * CHUNK, CHUNK)],
                         ems[b2], esem[b2])

    def _wait_in(ci, b2):
        pltpu.make_async_copy(hm_hbm.at[igat[b2]], rows[b2],
                              gsem[b2]).wait()
        pltpu.make_async_copy(em_hbm.at[pl.ds(cE + base_t + ci * CHUNK,
                                              CHUNK)],
                              ems[b2], esem[b2]).wait()

    def _wait_scat(b2, sl4):
        pltpu.make_async_copy(rows[b2], agg_sh.at[idst[sl4]],
                              ssem[b2]).wait()

    _start_idx(0, 0)
    _start_idx(1, 1)
    _start_idx(2, 2)
    _wait_idx(0, 0)
    _build_igat(0, 0)
    _start_in(0, 0)
    plsc.subcore_barrier()

    def _step(ci, b2, sl4):
        nsl = (sl4 + 1) % 4
        @pl.when(ci + 1 < N_CHUNKS)
        def _():
            _wait_idx(ci + 1, nsl)
            _build_igat(nsl, 1 - b2)
            @pl.when(ci >= 1)
            def _():
                _wait_scat(1 - b2, (sl4 + 3) % 4)
            _start_in(ci + 1, 1 - b2)
        @pl.when(ci + 3 < N_CHUNKS)
        def _():
            _start_idx(ci + 3, (sl4 + 3) % 4)
        _wait_in(ci, b2)

        @plsc.parallel_loop(0, CHUNK, unroll=2)
        def _row(r):
            for g in range(HH // 32):
                w = ems[b2][r, pl.ds(g * 16, 16)]
                flo = jax.lax.bitcast_convert_type(w << 16, _f32)
                fhi = jax.lax.bitcast_convert_type(w & jnp.int32(-65536),
                                                   _f32)
                for t, fv in ((0, flo), (1, fhi)):
                    sl = pl.ds(g * 32 + t * 16, 16)
                    u = rows[b2][r, sl] + fv
                    rows[b2][r, sl] = u * (1.0 / (1.0 + jnp.exp(-u)))
        pltpu.async_copy(rows[b2], agg_sh.at[idst[sl4]], ssem[b2],
                         add=True)

    def _quad(i, carry):
        ci = 4 * i
        _step(ci, 0, 0)
        _step(ci + 1, 1, 1)
        _step(ci + 2, 0, 2)
        _step(ci + 3, 1, 3)
        return carry
    lax.fori_loop(0, N_CHUNKS // 4, _quad, 0)
    _step(N_CHUNKS - 2, 0, 0)
    _step(N_CHUNKS - 1, 1, 1)

    _wait_scat(0, 0)
    _wait_scat(1, 1)
    plsc.subcore_barrier()

    def _wcopy(j, carry):
        off = s * OWN_ROWS + j * CHUNK
        pltpu.sync_copy(agg_sh.at[pl.ds(off, CHUNK)],
                        out_hbm.at[pl.ds(cV + off, CHUNK)])
        return carry
    lax.fori_loop(0, n_own, _wcopy, 0)


def _sc_aggregate(hm2, em2, src3, dst3):
    mesh = plsc.VectorSubcoreMesh(core_axis_name="c", subcore_axis_name="s")
    f = functools.partial(
        pl.kernel,
        mesh=mesh,
        out_type=jax.ShapeDtypeStruct((2 * V, HH), _f32),
        scratch_types=(
            [pltpu.VMEM((CHUNK, HH), _f32)] * 2
            + [pltpu.VMEM((CHUNK, HH // 2), jnp.int32)] * 2
            + [pltpu.VMEM((CHUNK,), jnp.int32)] * 10
            + [pltpu.SemaphoreType.DMA] * 10
            + [pltpu.VMEM_SHARED((V, HH), _f32)]
        ),
    )(_sc_body)
    return f(hm2, em2, src3, dst3)


def _update_body(h_ref, alo_ref, ahi_ref, wu_ref, bu_ref, g_ref, bt_ref,
                 wm_ref, bm_ref, h1_ref, hm_ref):
    upd = (jnp.dot(alo_ref[0], wu_ref[...][:HH, :],
                   preferred_element_type=_f32)
           + jnp.dot(ahi_ref[0], wu_ref[...][HH:, :],
                     preferred_element_type=_f32)
           + bu_ref[...])
    y = h_ref[...] + _silu(upd)
    mu = jnp.mean(y, axis=-1, keepdims=True)
    yc = y - mu
    var = jnp.mean(yc * yc, axis=-1, keepdims=True)
    h1 = yc * jax.lax.rsqrt(var + 1e-5) * g_ref[...] + bt_ref[...]
    h1_ref[...] = h1
    if hm_ref is not None:
        hm = jnp.dot(h1, wm_ref[...], preferred_element_type=_f32) + bm_ref[...]
        hm_ref[0] = hm[:, :HH]
        hm_ref[1] = hm[:, HH:]


def _update(h, agg2, Wu, bu, g, bt, Wm=None, bm=None):
    VB = 2000
    grid = (V // VB,)
    with_next = Wm is not None
    if not with_next:
        Wm = jnp.zeros((H, H), _f32)
        bm = jnp.zeros((H,), _f32)
    agg3 = agg2.reshape(2, V, HH)

    def body(h_ref, alo_ref, ahi_ref, wu_ref, bu_ref, g_ref, bt_ref,
             wm_ref, bm_ref, h1_ref, *maybe_hm):
        _update_body(h_ref, alo_ref, ahi_ref, wu_ref, bu_ref, g_ref, bt_ref,
                     wm_ref, bm_ref, h1_ref,
                     maybe_hm[0] if maybe_hm else None)

    out_specs = [pl.BlockSpec((VB, H), lambda i: (i, 0))]
    out_shape = [jax.ShapeDtypeStruct((V, H), _f32)]
    if with_next:
        out_specs.append(pl.BlockSpec((2, VB, HH), lambda i: (0, i, 0)))
        out_shape.append(jax.ShapeDtypeStruct((2, V, HH), _f32))

    res = pl.pallas_call(
        body,
        grid=grid,
        in_specs=[
            pl.BlockSpec((VB, H), lambda i: (i, 0)),
            pl.BlockSpec((1, VB, HH), lambda i: (0, i, 0)),
            pl.BlockSpec((1, VB, HH), lambda i: (1, i, 0)),
            pl.BlockSpec((H, H), lambda i: (0, 0)),
            pl.BlockSpec((1, H), lambda i: (0, 0)),
            pl.BlockSpec((1, H), lambda i: (0, 0)),
            pl.BlockSpec((1, H), lambda i: (0, 0)),
            pl.BlockSpec((H, H), lambda i: (0, 0)),
            pl.BlockSpec((1, H), lambda i: (0, 0)),
        ],
        out_specs=out_specs,
        out_shape=out_shape,
    )(h, agg3, agg3, Wu, bu.reshape(1, H), g.reshape(1, H), bt.reshape(1, H),
      Wm, bm.reshape(1, H))
    return res if with_next else (res[0], None)


def kernel(x, edge_index, edge_attr, W_node, b_node, W_edge, b_edge,
           W_m0, b_m0, W_u0, b_u0, ln_g0, ln_b0,
           W_m1, b_m1, W_u1, b_u1, ln_g1, ln_b1):
    src3 = edge_index[0]
    dst3 = edge_index[1]

    def _wm_sel(Wm):
        return tuple(Wm[:, sel].astype(_bf16) for sel in (LO0, HI0,
                                                          LO1, HI1))
    wm0s = _wm_sel(W_m0)
    wm1s = _wm_sel(W_m1)

    h, hm0 = _node_embed(x, W_node, b_node, W_m0, b_m0)
    em0 = _edge_messages(edge_attr, W_edge, b_edge, *wm0s)

    agg0 = _sc_aggregate(hm0.reshape(2 * V, HH),
                         em0.reshape(2 * E, HH // 2), src3, dst3)
    em1 = _edge_messages(edge_attr, W_edge, b_edge, *wm1s)
    h1, hm1 = _update(h, agg0, W_u0, b_u0, ln_g0, ln_b0, W_m1, b_m1)

    agg1 = _sc_aggregate(hm1.reshape(2 * V, HH),
                         em1.reshape(2 * E, HH // 2), src3, dst3)
    h2, _ = _update(h1, agg1, W_u1, b_u1, ln_g1, ln_b1)
    return h2


# speedup vs baseline: 1.0721x; 1.0324x over previous
import functools

import jax
import jax.numpy as jnp
import numpy as np
from jax import lax
from jax.experimental import pallas as pl
from jax.experimental.pallas import tpu as pltpu
from jax.experimental.pallas import tpu_sc as plsc

V = 10000
E = 320000
DN = 128
DE = 16
H = 256
HH = H // 2

N_TILES = 16
EDGES_PER_TILE = E // N_TILES
CHUNK = 80
N_CHUNKS = EDGES_PER_TILE // CHUNK
OWN_ROWS = 640
OWN_CHUNKS_FULL = OWN_ROWS // CHUNK
OWN_CHUNKS_LAST = (V - 15 * OWN_ROWS) // CHUNK

_f32 = jnp.float32
_bf16 = jnp.bfloat16

def _pack_cols(half, hi):
    return np.concatenate(
        [128 * half + 32 * g + 16 * hi + np.arange(16) for g in range(4)])

CAT = np.concatenate([_pack_cols(0, 0), _pack_cols(1, 0),
                      _pack_cols(0, 1), _pack_cols(1, 1)])


def _silu(x):
    return x * (1.0 / (1.0 + jnp.exp(-x)))


def _embed_body(x_ref, wn_ref, bn_ref, wm_ref, bm_ref, h_ref, hm_ref):
    h = _silu(jnp.dot(x_ref[...], wn_ref[...],
                      preferred_element_type=_f32) + bn_ref[...])
    h_ref[...] = h
    hm = jnp.dot(h, wm_ref[...], preferred_element_type=_f32) + bm_ref[...]
    hm_ref[0] = hm[:, :HH]
    hm_ref[1] = hm[:, HH:]


def _node_embed(x, Wn, bn, Wm, bm):
    VB = 2000
    grid = (V // VB,)
    return pl.pallas_call(
        _embed_body,
        grid=grid,
        in_specs=[
            pl.BlockSpec((VB, DN), lambda i: (i, 0)),
            pl.BlockSpec((DN, H), lambda i: (0, 0)),
            pl.BlockSpec((1, H), lambda i: (0, 0)),
            pl.BlockSpec((H, H), lambda i: (0, 0)),
            pl.BlockSpec((1, H), lambda i: (0, 0)),
        ],
        out_specs=[
            pl.BlockSpec((VB, H), lambda i: (i, 0)),
            pl.BlockSpec((2, VB, HH), lambda i: (0, i, 0)),
        ],
        out_shape=[
            jax.ShapeDtypeStruct((V, H), _f32),
            jax.ShapeDtypeStruct((2, V, HH), _f32),
        ],
    )(x, Wn, bn.reshape(1, H), Wm, bm.reshape(1, H))


def _bf16_bits(x):
    return jax.lax.bitcast_convert_type(
        x.astype(_bf16).astype(_f32), jnp.uint32)


def _pack_words(cat):
    return ((_bf16_bits(cat[:, HH:]) & jnp.uint32(0xFFFF0000)) | (
        _bf16_bits(cat[:, :HH]) >> 16)).astype(jnp.int32)


def _edge_body(ea_ref, we_ref, be_ref, wm_ref, em_ref):
    e = _silu(jnp.dot(ea_ref[...], we_ref[...],
                      preferred_element_type=_f32) + be_ref[...])
    w = _pack_words(jnp.dot(e.astype(_bf16), wm_ref[...],
                            preferred_element_type=_f32))
    em_ref[0] = w[:, :HH // 2]
    em_ref[1] = w[:, HH // 2:]


def _edge_messages(ea, We, be, Wm_cat):
    CE = 2000
    grid = (E // CE,)
    return pl.pallas_call(
        _edge_body,
        grid=grid,
        in_specs=[
            pl.BlockSpec((CE, DE), lambda i: (i, 0)),
            pl.BlockSpec((DE, H), lambda i: (0, 0)),
            pl.BlockSpec((1, H), lambda i: (0, 0)),
            pl.BlockSpec((H, H), lambda i: (0, 0)),
        ],
        out_specs=pl.BlockSpec((2, CE, HH // 2), lambda i: (0, i, 0)),
        out_shape=jax.ShapeDtypeStruct((2, E, HH // 2), jnp.int32),
    )(ea, We, be.reshape(1, H), Wm_cat)


def _sc_body(hm_hbm, em_hbm, src_hbm, dst_hbm, out_hbm,
             rows0, rows1, em0, em1,
             isrc0, isrc1, isrc2, isrc3, idst0, idst1, idst2, idst3,
             igat0, igat1,
             gsem0, gsem1, esem0, esem1, ssem0, ssem1,
             isem0, isem1, isem2, isem3, agg_sh):
    c = lax.axis_index("c")
    s = lax.axis_index("s")
    n_own = jnp.where(s == N_TILES - 1, OWN_CHUNKS_LAST, OWN_CHUNKS_FULL)
    cV = c * V
    cE = c * E
    base_t = s * EDGES_PER_TILE
    rows = (rows0, rows1)
    ems = (em0, em1)
    igat = (igat0, igat1)
    isrc = (isrc0, isrc1, isrc2, isrc3)
    idst = (idst0, idst1, idst2, idst3)
    gsem = (gsem0, gsem1)
    esem = (esem0, esem1)
    ssem = (ssem0, ssem1)
    isem = (isem0, isem1, isem2, isem3)

    def _zrow(r, carry):
        for k in range(HH // 16):
            rows0[r, pl.ds(k * 16, 16)] = jnp.zeros((16,), _f32)
        return carry
    lax.fori_loop(0, CHUNK, _zrow, 0)

    def _zcopy(j, carry):
        pltpu.sync_copy(rows0,
                        agg_sh.at[pl.ds(s * OWN_ROWS + j * CHUNK, CHUNK)])
        return carry
    lax.fori_loop(0, n_own, _zcopy, 0)

    def _start_idx(ci, sl4):
        eb = pl.ds(base_t + ci * CHUNK, CHUNK)
        pltpu.async_copy(src_hbm.at[eb], isrc[sl4], isem[sl4])
        pltpu.async_copy(dst_hbm.at[eb], idst[sl4], isem[sl4])

    def _wait_idx(ci, sl4):
        eb = pl.ds(base_t + ci * CHUNK, CHUNK)
        pltpu.make_async_copy(src_hbm.at[eb], isrc[sl4], isem[sl4]).wait()
        pltpu.make_async_copy(dst_hbm.at[eb], idst[sl4], isem[sl4]).wait()

    def _build_igat(sl4, b2):
        for k in range(CHUNK // 16):
            sl = pl.ds(k * 16, 16)
            igat[b2][sl] = isrc[sl4][sl] + cV

    def _start_in(ci, b2):
        pltpu.async_copy(hm_hbm.at[igat[b2]], rows[b2], gsem[b2])
        pltpu.async_copy(em_hbm.at[pl.ds(cE + base_t + ci * CHUNK, CHUNK)],
                         ems[b2], esem[b2])

    def _wait_in(ci, b2):
        pltpu.make_async_copy(hm_hbm.at[igat[b2]], rows[b2],
                              gsem[b2]).wait()
        pltpu.make_async_copy(em_hbm.at[pl.ds(cE + base_t + ci * CHUNK,
                                              CHUNK)],
                              ems[b2], esem[b2]).wait()

    def _wait_scat(b2, sl4):
        pltpu.make_async_copy(rows[b2], agg_sh.at[idst[sl4]],
                              ssem[b2]).wait()

    _start_idx(0, 0)
    _start_idx(1, 1)
    _start_idx(2, 2)
    _wait_idx(0, 0)
    _build_igat(0, 0)
    _start_in(0, 0)
    plsc.subcore_barrier()

    def _step(ci, b2, sl4):
        nsl = (sl4 + 1) % 4
        @pl.when(ci + 1 < N_CHUNKS)
        def _():
            _wait_idx(ci + 1, nsl)
            _build_igat(nsl, 1 - b2)
            @pl.when(ci >= 1)
            def _():
                _wait_scat(1 - b2, (sl4 + 3) % 4)
            _start_in(ci + 1, 1 - b2)
        @pl.when(ci + 3 < N_CHUNKS)
        def _():
            _start_idx(ci + 3, (sl4 + 3) % 4)
        _wait_in(ci, b2)

        @plsc.parallel_loop(0, CHUNK, unroll=2)
        def _row(r):
            for g in range(HH // 32):
                we = ems[b2][r, pl.ds(g * 16, 16)]
                elo = jax.lax.bitcast_convert_type(we << 16, _f32)
                ehi = jax.lax.bitcast_convert_type(we & jnp.int32(-65536),
                                                   _f32)
                for t, ev in ((0, elo), (1, ehi)):
                    sl = pl.ds(g * 32 + t * 16, 16)
                    u = rows[b2][r, sl] + ev
                    rows[b2][r, sl] = u * (1.0 / (1.0 + jnp.exp(-u)))
        pltpu.async_copy(rows[b2], agg_sh.at[idst[sl4]], ssem[b2],
                         add=True)

    def _quad(i, carry):
        ci = 4 * i
        _step(ci, 0, 0)
        _step(ci + 1, 1, 1)
        _step(ci + 2, 0, 2)
        _step(ci + 3, 1, 3)
        return carry
    lax.fori_loop(0, N_CHUNKS // 4, _quad, 0)
    _step(N_CHUNKS - 2, 0, 0)
    _step(N_CHUNKS - 1, 1, 1)

    _wait_scat(0, 0)
    _wait_scat(1, 1)
    plsc.subcore_barrier()

    def _wcopy(j, carry):
        off = s * OWN_ROWS + j * CHUNK
        pltpu.sync_copy(agg_sh.at[pl.ds(off, CHUNK)],
                        out_hbm.at[pl.ds(cV + off, CHUNK)])
        return carry
    lax.fori_loop(0, n_own, _wcopy, 0)


def _sc_aggregate(hm2, em2, src3, dst3):
    mesh = plsc.VectorSubcoreMesh(core_axis_name="c", subcore_axis_name="s")
    f = functools.partial(
        pl.kernel,
        mesh=mesh,
        out_type=jax.ShapeDtypeStruct((2 * V, HH), _f32),
        scratch_types=(
            [pltpu.VMEM((CHUNK, HH), _f32)] * 2
            + [pltpu.VMEM((CHUNK, HH // 2), jnp.int32)] * 2
            + [pltpu.VMEM((CHUNK,), jnp.int32)] * 10
            + [pltpu.SemaphoreType.DMA] * 10
            + [pltpu.VMEM_SHARED((V, HH), _f32)]
        ),
    )(_sc_body)
    return f(hm2, em2, src3, dst3)


def _update_body(h_ref, alo_ref, ahi_ref, wu_ref, bu_ref, g_ref, bt_ref,
                 wm_ref, bm_ref, h1_ref, hm_ref):
    upd = (jnp.dot(alo_ref[0], wu_ref[...][:HH, :],
                   preferred_element_type=_f32)
           + jnp.dot(ahi_ref[0], wu_ref[...][HH:, :],
                     preferred_element_type=_f32)
           + bu_ref[...])
    y = h_ref[...] + _silu(upd)
    mu = jnp.mean(y, axis=-1, keepdims=True)
    yc = y - mu
    var = jnp.mean(yc * yc, axis=-1, keepdims=True)
    h1 = yc * jax.lax.rsqrt(var + 1e-5) * g_ref[...] + bt_ref[...]
    h1_ref[...] = h1
    if hm_ref is not None:
        hm = jnp.dot(h1, wm_ref[...],
                     preferred_element_type=_f32) + bm_ref[...]
        hm_ref[0] = hm[:, :HH]
        hm_ref[1] = hm[:, HH:]


def _update(h, agg2, Wu, bu, g, bt, Wm=None, bm=None):
    VB = 2000
    grid = (V // VB,)
    with_next = Wm is not None
    if not with_next:
        Wm = jnp.zeros((H, H), _f32)
        bm = jnp.zeros((H,), _f32)
    agg3 = agg2.reshape(2, V, HH)

    def body(h_ref, alo_ref, ahi_ref, wu_ref, bu_ref, g_ref, bt_ref,
             wm_ref, bm_ref, h1_ref, *maybe_hm):
        _update_body(h_ref, alo_ref, ahi_ref, wu_ref, bu_ref, g_ref, bt_ref,
                     wm_ref, bm_ref, h1_ref,
                     maybe_hm[0] if maybe_hm else None)

    out_specs = [pl.BlockSpec((VB, H), lambda i: (i, 0))]
    out_shape = [jax.ShapeDtypeStruct((V, H), _f32)]
    if with_next:
        out_specs.append(pl.BlockSpec((2, VB, HH), lambda i: (0, i, 0)))
        out_shape.append(jax.ShapeDtypeStruct((2, V, HH), _f32))

    res = pl.pallas_call(
        body,
        grid=grid,
        in_specs=[
            pl.BlockSpec((VB, H), lambda i: (i, 0)),
            pl.BlockSpec((1, VB, HH), lambda i: (0, i, 0)),
            pl.BlockSpec((1, VB, HH), lambda i: (1, i, 0)),
            pl.BlockSpec((H, H), lambda i: (0, 0)),
            pl.BlockSpec((1, H), lambda i: (0, 0)),
            pl.BlockSpec((1, H), lambda i: (0, 0)),
            pl.BlockSpec((1, H), lambda i: (0, 0)),
            pl.BlockSpec((H, H), lambda i: (0, 0)),
            pl.BlockSpec((1, H), lambda i: (0, 0)),
        ],
        out_specs=out_specs,
        out_shape=out_shape,
    )(h, agg3, agg3, Wu, bu.reshape(1, H), g.reshape(1, H), bt.reshape(1, H),
      Wm, bm.reshape(1, H))
    return res if with_next else (res[0], None)


def kernel(x, edge_index, edge_attr, W_node, b_node, W_edge, b_edge,
           W_m0, b_m0, W_u0, b_u0, ln_g0, ln_b0,
           W_m1, b_m1, W_u1, b_u1, ln_g1, ln_b1):
    src3 = edge_index[0]
    dst3 = edge_index[1]

    Wm0cb = W_m0[:, CAT].astype(_bf16)
    Wm1cb = W_m1[:, CAT].astype(_bf16)

    h, hm0 = _node_embed(x, W_node, b_node, W_m0, b_m0)
    em0 = _edge_messages(edge_attr, W_edge, b_edge, Wm0cb)

    agg0 = _sc_aggregate(hm0.reshape(2 * V, HH),
                         em0.reshape(2 * E, HH // 2), src3, dst3)
    em1 = _edge_messages(edge_attr, W_edge, b_edge, Wm1cb)
    h1, hm1 = _update(h, agg0, W_u0, b_u0, ln_g0, ln_b0, W_m1, b_m1)

    agg1 = _sc_aggregate(hm1.reshape(2 * V, HH),
                         em1.reshape(2 * E, HH // 2), src3, dst3)
    h2, _ = _update(h1, agg1, W_u1, b_u1, ln_g1, ln_b1)
    return h2
